# baseline (speedup 1.0000x reference)
import functools

import jax
import jax.numpy as jnp
from jax import lax
from jax.experimental import pallas as pl
from jax.experimental.pallas import tpu as pltpu
from jax.experimental.pallas import tpu_sc as plsc

B, N, H = 4096, 200, 128
ROWS = B * N
NW = 32
ROWS_PER_W = ROWS // NW
STEP = 128
NSTEPS = ROWS_PER_W // STEP


def _sc_gather(table, idx):
    mesh = plsc.VectorSubcoreMesh(core_axis_name="c", subcore_axis_name="s")

    @functools.partial(
        pl.kernel,
        mesh=mesh,
        out_type=jax.ShapeDtypeStruct((ROWS, H), jnp.float32),
        scratch_types=[
            pltpu.VMEM((NSTEPS, STEP), jnp.int32),
            pltpu.VMEM((STEP, H), jnp.float32),
            pltpu.SemaphoreType.DMA,
        ],
    )
    def k(table_hbm, idx_hbm, out_hbm, idx_v, rows_v, sem):
        wid = lax.axis_index("s") * 2 + lax.axis_index("c")
        base = wid * ROWS_PER_W
        pltpu.sync_copy(idx_hbm.at[wid], idx_v)

        def body(j, carry):
            pltpu.async_copy(table_hbm.at[idx_v.at[j]], rows_v, sem).wait()
            pltpu.sync_copy(rows_v, out_hbm.at[pl.ds(base + j * STEP, STEP)])
            return carry

        lax.fori_loop(0, NSTEPS, body, 0)

    return k(table, idx)


def kernel(points, labels, point_embeddings_weight):
    del points
    idx = labels.astype(jnp.int32).reshape(NW, NSTEPS, STEP)
    out = _sc_gather(point_embeddings_weight, idx)
    return out.reshape(B, N, H)


# speedup vs baseline: 66.1413x; 66.1413x over previous
import functools

import jax
import jax.numpy as jnp
from jax import lax
from jax.experimental import pallas as pl
from jax.experimental.pallas import tpu as pltpu
from jax.experimental.pallas import tpu_sc as plsc

B, N, H = 4096, 200, 128
ROWS = B * N
NW = 32
ROWS_PER_W = ROWS // NW
STEP = 128
NSTEPS = ROWS_PER_W // STEP


def _sc_gather(table, idx):
    mesh = plsc.VectorSubcoreMesh(core_axis_name="c", subcore_axis_name="s")

    @functools.partial(
        pl.kernel,
        mesh=mesh,
        out_type=jax.ShapeDtypeStruct((ROWS, H), jnp.float32),
        scratch_types=[
            pltpu.VMEM((NSTEPS, STEP), jnp.int32),
            pltpu.VMEM((STEP, H), jnp.float32),
            pltpu.VMEM((STEP, H), jnp.float32),
            pltpu.VMEM_SHARED((4, H), jnp.float32),
            pltpu.SemaphoreType.DMA,
            pltpu.SemaphoreType.DMA,
            pltpu.SemaphoreType.DMA,
        ],
    )
    def k(table_hbm, idx_hbm, out_hbm, idx_v, buf0, buf1, table_s,
          sem_g, sem_o0, sem_o1):
        wid = lax.axis_index("s") * 2 + lax.axis_index("c")
        base = wid * ROWS_PER_W

        @pl.when(lax.axis_index("s") == 0)
        def _():
            pltpu.sync_copy(table_hbm, table_s)

        pltpu.sync_copy(idx_hbm.at[wid], idx_v)
        plsc.subcore_barrier()

        def out_slice(j):
            return out_hbm.at[pl.ds(base + j * STEP, STEP)]

        def fire_gather(j, buf):
            pltpu.async_copy(table_s.at[idx_v.at[j]], buf, sem_g)

        def wait_gather(buf):
            pltpu.make_async_copy(table_s.at[idx_v.at[0]], buf, sem_g).wait()

        def fire_out(j, buf, sem):
            pltpu.async_copy(buf, out_slice(j), sem)

        def wait_out(buf, sem):
            pltpu.make_async_copy(buf, out_slice(0), sem).wait()

        fire_gather(0, buf0)

        def body(t, carry):
            a = 2 * t
            wait_gather(buf0)
            fire_out(a, buf0, sem_o0)

            @pl.when(t >= 1)
            def _():
                wait_out(buf1, sem_o1)

            fire_gather(a + 1, buf1)
            wait_gather(buf1)
            fire_out(a + 1, buf1, sem_o1)

            @pl.when(t < NSTEPS // 2 - 1)
            def _():
                wait_out(buf0, sem_o0)
                fire_gather(a + 2, buf0)

            return carry

        lax.fori_loop(0, NSTEPS // 2, body, 0)
        wait_out(buf0, sem_o0)
        wait_out(buf1, sem_o1)

    return k(table, idx)


def kernel(points, labels, point_embeddings_weight):
    del points
    idx = labels.astype(jnp.int32).reshape(NW, NSTEPS, STEP)
    out = _sc_gather(point_embeddings_weight, idx)
    return out.reshape(B, N, H)
